# Initial kernel scaffold; baseline (speedup 1.0000x reference)
import jax
import jax.numpy as jnp
from jax.experimental import pallas as pl


def kernel(x, edge_index, edge_weight, W1, b1, W2, b2, W3, b3):
    raise NotImplementedError("write your pallas kernel here")


# speedup vs baseline: 5.4909x; 5.4909x over previous
import jax
import jax.numpy as jnp
from jax import lax
from jax.experimental import pallas as pl
from jax.experimental.pallas import tpu as pltpu
from jax.experimental.pallas import tpu_sc as plsc

_NC = 2
_NS = 16
_NW = _NC * _NS
_K = 80
_CH = 125
_ZR = 125


def _spmm_partials(h, col_r, row_r, w_r):
    n, d = h.shape
    rows_per_tile = n // _NS
    nsplat = d // 16
    mesh = plsc.VectorSubcoreMesh(core_axis_name="c", subcore_axis_name="s")

    def body(h_hbm, col_hbm, row_hbm, w_hbm, out_hbm,
             col_v, row_v, w_v, rows_v, acc):
        c = lax.axis_index("c")
        s = lax.axis_index("s")
        wid = c * _NS + s

        pltpu.sync_copy(col_hbm.at[wid], col_v)
        pltpu.sync_copy(row_hbm.at[wid], row_v)
        pltpu.sync_copy(w_hbm.at[wid], w_v)

        zz = jnp.zeros((16,), jnp.float32)

        def zbody(i, carry):
            for k in range(nsplat):
                rows_v[i, pl.ds(16 * k, 16)] = zz
            return carry

        lax.fori_loop(0, _K, zbody, 0)
        base = s * rows_per_tile
        nfull, rem = divmod(rows_per_tile, _K)
        for t in range(nfull):
            pltpu.sync_copy(rows_v, acc.at[pl.ds(base + t * _K, _K)])
        if rem:
            pltpu.sync_copy(rows_v.at[pl.ds(0, rem)],
                            acc.at[pl.ds(base + nfull * _K, rem)])
        plsc.subcore_barrier()

        def chunk(j, carry):
            pltpu.sync_copy(h_hbm.at[col_v.at[j]], rows_v)

            jbase = j * _K

            def ebody(e, ecarry):
                ids = lax.broadcast_in_dim(jbase + e, (16,), ())
                wb = plsc.load_gather(w_v, [ids])
                for k in range(nsplat):
                    sl = pl.ds(16 * k, 16)
                    rows_v[e, sl] = rows_v[e, sl] * wb
                return ecarry

            lax.fori_loop(0, _K, ebody, 0)
            pltpu.sync_copy(rows_v, acc.at[row_v.at[j]], add=True)
            return carry

        lax.fori_loop(0, _CH, chunk, 0)
        plsc.subcore_barrier()

        pltpu.sync_copy(acc.at[pl.ds(base, rows_per_tile)],
                        out_hbm.at[c, pl.ds(base, rows_per_tile)])

    return pl.kernel(
        body,
        out_type=jax.ShapeDtypeStruct((_NC, n, d), jnp.float32),
        mesh=mesh,
        compiler_params=pltpu.CompilerParams(use_tc_tiling_on_sc=False,
                                             needs_layout_passes=False),
        scratch_types=[
            pltpu.VMEM((_CH, _K), jnp.int32),
            pltpu.VMEM((_CH, _K), jnp.int32),
            pltpu.VMEM((_CH * _K,), jnp.float32),
            pltpu.VMEM((_K, d), jnp.float32),
            pltpu.VMEM_SHARED((n, d), jnp.float32),
        ],
    )(h, col_r, row_r, w_r)


def _dense(p, W, b, elu_sum):
    d = p.shape[-1]
    n = p.shape[-2]
    blk = 1000
    grid = (n // blk,)
    b2d = b.reshape(1, d)

    if elu_sum:
        def body(p_ref, w_ref, b_ref, o_ref):
            sacc = p_ref[0] + p_ref[1]
            hh = jnp.where(sacc > 0, sacc, jnp.exp(jnp.minimum(sacc, 0.0)) - 1.0)
            o_ref[...] = (jnp.dot(hh, w_ref[...],
                                  preferred_element_type=jnp.float32)
                          + b_ref[...])
        in_specs = [
            pl.BlockSpec((_NC, blk, d), lambda i: (0, i, 0)),
            pl.BlockSpec((d, d), lambda i: (0, 0)),
            pl.BlockSpec((1, d), lambda i: (0, 0)),
        ]
    else:
        def body(p_ref, w_ref, b_ref, o_ref):
            o_ref[...] = (jnp.dot(p_ref[...], w_ref[...],
                                  preferred_element_type=jnp.float32)
                          + b_ref[...])
        in_specs = [
            pl.BlockSpec((blk, d), lambda i: (i, 0)),
            pl.BlockSpec((d, d), lambda i: (0, 0)),
            pl.BlockSpec((1, d), lambda i: (0, 0)),
        ]

    return pl.pallas_call(
        body,
        grid=grid,
        in_specs=in_specs,
        out_specs=pl.BlockSpec((blk, d), lambda i: (i, 0)),
        out_shape=jax.ShapeDtypeStruct((n, d), jnp.float32),
    )(p, W, b2d)


def kernel(x, edge_index, edge_weight, W1, b1, W2, b2, W3, b3):
    row = edge_index[0].astype(jnp.int32).reshape(_NW, _CH, _K)
    col = edge_index[1].astype(jnp.int32).reshape(_NW, _CH, _K)
    w_r = edge_weight.reshape(_NW, _CH * _K)

    h0 = _dense(x, W1, b1, False)
    a0 = _spmm_partials(h0, col, row, w_r)
    h1 = _dense(a0, W2, b2, True)
    a1 = _spmm_partials(h1, col, row, w_r)
    return _dense(a1, W3, b3, True)


# speedup vs baseline: 12.3624x; 2.2514x over previous
import jax
import jax.numpy as jnp
from jax import lax
from jax.experimental import pallas as pl
from jax.experimental.pallas import tpu as pltpu
from jax.experimental.pallas import tpu_sc as plsc

_NC = 2
_NS = 16
_NW = _NC * _NS
_K = 40
_CH = 250
_NB = 4


def _spmm_partials(h, col_r, row_r, w_r):
    n, d = h.shape
    rows_per_tile = n // _NS
    nsplat = d // 16
    mesh = plsc.VectorSubcoreMesh(core_axis_name="c", subcore_axis_name="s")

    def body(h_hbm, col_hbm, row_hbm, w_hbm, out_hbm,
             col_v, row_v, w_v, r0, r1, r2, r3,
             g0, g1, g2, g3, s0, s1, s2, s3, acc):
        rows_bufs = (r0, r1, r2, r3)
        gsems = (g0, g1, g2, g3)
        ssems = (s0, s1, s2, s3)
        c_ax = lax.axis_index("c")
        s_ax = lax.axis_index("s")
        wid = c_ax * _NS + s_ax

        pltpu.sync_copy(col_hbm.at[wid], col_v)
        pltpu.sync_copy(row_hbm.at[wid], row_v)
        pltpu.sync_copy(w_hbm.at[wid], w_v)

        zz = jnp.zeros((16,), jnp.float32)

        def zbody(i, carry):
            for k in range(nsplat):
                r0[i, pl.ds(16 * k, 16)] = zz
            return carry

        lax.fori_loop(0, _K, zbody, 0)
        base = s_ax * rows_per_tile
        nfull, rem = divmod(rows_per_tile, _K)
        for t in range(nfull):
            pltpu.sync_copy(r0, acc.at[pl.ds(base + t * _K, _K)])
        if rem:
            pltpu.sync_copy(r0.at[pl.ds(0, rem)],
                            acc.at[pl.ds(base + nfull * _K, rem)])
        plsc.subcore_barrier()

        def issue_gather(c, b):
            return pltpu.async_copy(h_hbm.at[col_v.at[c]], rows_bufs[b],
                                    gsems[b])

        def wait_gather(c, b):
            pltpu.make_async_copy(h_hbm.at[col_v.at[c]], rows_bufs[b],
                                  gsems[b]).wait()

        def issue_scatter(c, b):
            return pltpu.async_copy(rows_bufs[b], acc.at[row_v.at[c]],
                                    ssems[b], add=True)

        def wait_scatter(c, b):
            pltpu.make_async_copy(rows_bufs[b], acc.at[row_v.at[c]],
                                  ssems[b]).wait()

        def scale(c, b):
            rows = rows_bufs[b]
            jbase = c * _K

            @plsc.parallel_loop(0, _K, unroll=4)
            def _(e):
                ids = lax.broadcast_in_dim(jbase + e, (16,), ())
                wb = plsc.load_gather(w_v, [ids])
                for k in range(nsplat):
                    sl = pl.ds(16 * k, 16)
                    rows[e, sl] = rows[e, sl] * wb

        def chunk_body(c, b, wait_prev, next_c):
            wait_gather(c, b)
            scale(c, b)
            issue_scatter(c, b)
            bp = (b + _NB - 1) % _NB
            if wait_prev:
                wait_scatter(c - 1, bp)
            if next_c is not None:
                issue_gather(next_c, bp)

        issue_gather(0, 0)
        issue_gather(1, 1)
        issue_gather(2, 2)

        chunk_body(0, 0, False, 3)
        chunk_body(1, 1, True, 4)
        chunk_body(2, 2, True, 5)
        chunk_body(3, 3, True, 6)

        def group(g, carry):
            c0 = g * _NB
            for b in range(_NB):
                c = c0 + b
                chunk_body(c, b, True, jnp.minimum(c + 3, _CH - 1))
            return carry

        lax.fori_loop(1, (_CH - 2) // _NB, group, 0)

        chunk_body(_CH - 2, (_CH - 2) % _NB, True, None)
        chunk_body(_CH - 1, (_CH - 1) % _NB, True, None)
        wait_scatter(_CH - 1, (_CH - 1) % _NB)
        wait_gather(_CH - 1, (_CH - 4) % _NB)

        plsc.subcore_barrier()

        pltpu.sync_copy(acc.at[pl.ds(base, rows_per_tile)],
                        out_hbm.at[c_ax, pl.ds(base, rows_per_tile)])

    return pl.kernel(
        body,
        out_type=jax.ShapeDtypeStruct((_NC, n, d), jnp.float32),
        mesh=mesh,
        compiler_params=pltpu.CompilerParams(use_tc_tiling_on_sc=False,
                                             needs_layout_passes=False),
        scratch_types=[
            pltpu.VMEM((_CH, _K), jnp.int32),
            pltpu.VMEM((_CH, _K), jnp.int32),
            pltpu.VMEM((_CH * _K,), jnp.float32),
            pltpu.VMEM((_K, d), jnp.float32),
            pltpu.VMEM((_K, d), jnp.float32),
            pltpu.VMEM((_K, d), jnp.float32),
            pltpu.VMEM((_K, d), jnp.float32),
            pltpu.SemaphoreType.DMA,
            pltpu.SemaphoreType.DMA,
            pltpu.SemaphoreType.DMA,
            pltpu.SemaphoreType.DMA,
            pltpu.SemaphoreType.DMA,
            pltpu.SemaphoreType.DMA,
            pltpu.SemaphoreType.DMA,
            pltpu.SemaphoreType.DMA,
            pltpu.VMEM_SHARED((n, d), jnp.float32),
        ],
    )(h, col_r, row_r, w_r)


def _dense(p, W, b, elu_sum):
    d = p.shape[-1]
    n = p.shape[-2]
    blk = 1000
    grid = (n // blk,)
    b2d = b.reshape(1, d)

    if elu_sum:
        def body(p_ref, w_ref, b_ref, o_ref):
            sacc = p_ref[0] + p_ref[1]
            hh = jnp.where(sacc > 0, sacc, jnp.exp(jnp.minimum(sacc, 0.0)) - 1.0)
            o_ref[...] = (jnp.dot(hh, w_ref[...],
                                  preferred_element_type=jnp.float32)
                          + b_ref[...])
        in_specs = [
            pl.BlockSpec((_NC, blk, d), lambda i: (0, i, 0)),
            pl.BlockSpec((d, d), lambda i: (0, 0)),
            pl.BlockSpec((1, d), lambda i: (0, 0)),
        ]
    else:
        def body(p_ref, w_ref, b_ref, o_ref):
            o_ref[...] = (jnp.dot(p_ref[...], w_ref[...],
                                  preferred_element_type=jnp.float32)
                          + b_ref[...])
        in_specs = [
            pl.BlockSpec((blk, d), lambda i: (i, 0)),
            pl.BlockSpec((d, d), lambda i: (0, 0)),
            pl.BlockSpec((1, d), lambda i: (0, 0)),
        ]

    return pl.pallas_call(
        body,
        grid=grid,
        in_specs=in_specs,
        out_specs=pl.BlockSpec((blk, d), lambda i: (i, 0)),
        out_shape=jax.ShapeDtypeStruct((n, d), jnp.float32),
    )(p, W, b2d)


def kernel(x, edge_index, edge_weight, W1, b1, W2, b2, W3, b3):
    row = edge_index[0].astype(jnp.int32).reshape(_NW, _CH, _K)
    col = edge_index[1].astype(jnp.int32).reshape(_NW, _CH, _K)
    w_r = edge_weight.reshape(_NW, _CH * _K)

    h0 = _dense(x, W1, b1, False)
    a0 = _spmm_partials(h0, col, row, w_r)
    h1 = _dense(a0, W2, b2, True)
    a1 = _spmm_partials(h1, col, row, w_r)
    return _dense(a1, W3, b3, True)
